# baseline (speedup 1.0000x reference)
import functools

import jax
import jax.numpy as jnp
from jax import lax
from jax.experimental import pallas as pl
from jax.experimental.pallas import tpu as pltpu
from jax.experimental.pallas import tpu_sc as plsc

NC, NS, L = 2, 16, 16
NW = NC * NS
B = 65536
KEPT = 32768
D = 768
RPW = B // NW
K = 64


def _restore_kernel(outputs_hbm, order_hbm, out_hbm, ord_v, src_v, scale_v,
                    rows_v, sem):
    wid = lax.axis_index("s") * NC + lax.axis_index("c")
    base = wid * RPW
    pltpu.sync_copy(order_hbm.at[pl.ds(base, RPW)], ord_v)

    def chunk_body(c, _):
        def vec_body(q, _):
            o = ord_v[pl.ds(c * K + q * L, L)]
            pos = base + c * K + q * L + lax.iota(jnp.int32, 16)
            valid = o < KEPT
            src = jnp.where(valid, o, pos & (KEPT - 1))
            scl = jnp.where(valid, jnp.float32(2.0), jnp.float32(0.0))
            src_v[pl.ds(q * L, L)] = src
            scale_v[pl.ds(q * L, L)] = scl
            return None

        lax.fori_loop(0, K // L, vec_body, None)
        pltpu.async_copy(outputs_hbm.at[src_v], rows_v, sem).wait()

        def row_body(r, _):
            s = scale_v[pl.ds(r, L)][0]

            def col_body(kk, _):
                rows_v[r, pl.ds(kk * L, L)] = rows_v[r, pl.ds(kk * L, L)] * s
                return None

            lax.fori_loop(0, D // L, col_body, None)
            return None

        lax.fori_loop(0, K, row_body, None)
        pltpu.sync_copy(rows_v, out_hbm.at[pl.ds(base + c * K, K)])
        return None

    lax.fori_loop(0, RPW // K, chunk_body, None)


def kernel(outputs, indices):
    order = jnp.argsort(indices)
    mesh = plsc.VectorSubcoreMesh(core_axis_name="c", subcore_axis_name="s")
    f = functools.partial(
        pl.kernel,
        out_type=jax.ShapeDtypeStruct((B, D), jnp.float32),
        mesh=mesh,
        scratch_types=[
            pltpu.VMEM((RPW,), jnp.int32),
            pltpu.VMEM((K,), jnp.int32),
            pltpu.VMEM((K + L,), jnp.float32),
            pltpu.VMEM((K, D), jnp.float32),
            pltpu.SemaphoreType.DMA,
        ],
    )(_restore_kernel)
    return f(outputs, order)


# speedup vs baseline: 1.8555x; 1.8555x over previous
import functools

import jax
import jax.numpy as jnp
from jax import lax
from jax.experimental import pallas as pl
from jax.experimental.pallas import tpu as pltpu
from jax.experimental.pallas import tpu_sc as plsc

NC, NS, L = 2, 16, 16
NW = NC * NS
B = 65536
KEPT = 32768
D = 768
BINS = B // NW
CHUNK = 4096
K = 64
HCAP = 3072 + 5 * L


def _restore_kernel(outputs_hbm, indices_hbm, out_hbm,
                    idx_v, hitv, hitj, gjl, grl, zrl, hist_t, rows_v,
                    stg_g, stg_s, sem_g, sem_s):
    wid = lax.axis_index("s") * NC + lax.axis_index("c")
    lo = wid * BINS
    hi = lo + BINS
    lane = lax.iota(jnp.int32, L)
    lane0 = lane == 0
    zero16 = jnp.zeros((L,), jnp.int32)
    one16 = zero16 + 1
    onehot = jnp.where(lane0, jnp.int32(1), jnp.int32(0))

    def zh(b, _):
        hist_t[pl.ds(b * L, L)] = zero16
        return None

    lax.fori_loop(0, (BINS + L) // L, zh, None)

    def chunk_body(ci, carry):
        pltpu.sync_copy(indices_hbm.at[pl.ds(ci * CHUNK, CHUNK)], idx_v)

        def vec_body(i, carry2):
            nh, nbase = carry2
            v = idx_v[pl.ds(i * L, L)]
            nbase = nbase + jnp.sum(jnp.where(v < lo, one16, zero16))
            m = plsc.bitcast(v - lo, jnp.uint32) < jnp.uint32(BINS)
            cnt = jnp.sum(jnp.where(m, one16, zero16))

            @pl.when(cnt > 0)
            def _():
                jv = ci * CHUNK + i * L + lane
                plsc.store_compressed(hitv.at[pl.ds(nh, L)], v, mask=m)
                plsc.store_compressed(hitj.at[pl.ds(nh, L)], jv, mask=m)

            return (nh + cnt, nbase)

        return lax.fori_loop(0, CHUNK // L, vec_body, carry)

    nh, nbase = lax.fori_loop(0, B // CHUNK, chunk_body,
                              (jnp.int32(0), jnp.int32(0)))

    def hb(t, _):
        v = hitv[pl.ds(t, L)][0]
        b = v - lo
        tv = hist_t[pl.ds(b, L)]
        hist_t[pl.ds(b, L)] = tv + onehot
        return None

    lax.fori_loop(0, nh, hb, None)

    def pf(b, carry):
        h = hist_t[pl.ds(b * L, L)]
        c = plsc.cumsum(h)
        hist_t[pl.ds(b * L, L)] = carry + c - h
        return carry + c[L - 1]

    lax.fori_loop(0, BINS // L, pf, nbase)

    def ra(t, carry):
        nv, nz = carry
        v = hitv[pl.ds(t, L)][0]
        j = hitj[pl.ds(t, L)][0]
        b = v - lo
        tv = hist_t[pl.ds(b, L)]
        r = tv[0]
        hist_t[pl.ds(b, L)] = tv + onehot
        val = j < KEPT

        @pl.when(val)
        def _():
            og = gjl[pl.ds(nv, L)]
            gjl[pl.ds(nv, L)] = jnp.where(lane0, j, og)
            orr = grl[pl.ds(nv, L)]
            grl[pl.ds(nv, L)] = jnp.where(lane0, r, orr)

        @pl.when(jnp.logical_not(val))
        def _():
            oz = zrl[pl.ds(nz, L)]
            zrl[pl.ds(nz, L)] = jnp.where(lane0, r, oz)

        vi = val.astype(jnp.int32)
        return (nv + vi, nz + 1 - vi)

    nv, nz = lax.fori_loop(0, nh, ra, (jnp.int32(0), jnp.int32(0)))

    @pl.when(nv > 0)
    def _():
        jl = zero16 + gjl[pl.ds(nv - 1, L)][0]
        rl = zero16 + grl[pl.ds(nv - 1, L)][0]
        for q in range(K // L):
            gjl[pl.ds(nv + q * L, L)] = jl
            grl[pl.ds(nv + q * L, L)] = rl

    @pl.when(nz > 0)
    def _():
        zl = zero16 + zrl[pl.ds(nz - 1, L)][0]
        for q in range(K // L):
            zrl[pl.ds(nz + q * L, L)] = zl

    ncv = (nv + K - 1) // K
    ncz = (nz + K - 1) // K

    def mv(c, _):
        for q in range(K // L):
            stg_g[pl.ds(q * L, L)] = gjl[pl.ds(c * K + q * L, L)]
        pltpu.async_copy(outputs_hbm.at[stg_g], rows_v, sem_g).wait()

        def dbl(r, _):
            for k in range(D // L):
                x = rows_v[r, pl.ds(k * L, L)]
                rows_v[r, pl.ds(k * L, L)] = x + x
            return None

        lax.fori_loop(0, K, dbl, None)
        for q in range(K // L):
            stg_s[pl.ds(q * L, L)] = grl[pl.ds(c * K + q * L, L)]
        pltpu.async_copy(rows_v, out_hbm.at[stg_s], sem_s).wait()
        return None

    lax.fori_loop(0, ncv, mv, None)

    zf = jnp.zeros((L,), jnp.float32)

    def zb(r, _):
        for k in range(D // L):
            rows_v[r, pl.ds(k * L, L)] = zf
        return None

    lax.fori_loop(0, K, zb, None)

    def mz(c, _):
        for q in range(K // L):
            stg_s[pl.ds(q * L, L)] = zrl[pl.ds(c * K + q * L, L)]
        pltpu.async_copy(rows_v, out_hbm.at[stg_s], sem_s).wait()
        return None

    lax.fori_loop(0, ncz, mz, None)


def kernel(outputs, indices):
    mesh = plsc.VectorSubcoreMesh(core_axis_name="c", subcore_axis_name="s")
    f = functools.partial(
        pl.kernel,
        out_type=jax.ShapeDtypeStruct((B, D), jnp.float32),
        mesh=mesh,
        compiler_params=pltpu.CompilerParams(needs_layout_passes=False),
        scratch_types=[
            pltpu.VMEM((CHUNK,), jnp.int32),
            pltpu.VMEM((HCAP,), jnp.int32),
            pltpu.VMEM((HCAP,), jnp.int32),
            pltpu.VMEM((HCAP,), jnp.int32),
            pltpu.VMEM((HCAP,), jnp.int32),
            pltpu.VMEM((HCAP,), jnp.int32),
            pltpu.VMEM((BINS + L,), jnp.int32),
            pltpu.VMEM((K, D), jnp.float32),
            pltpu.VMEM((K,), jnp.int32),
            pltpu.VMEM((K,), jnp.int32),
            pltpu.SemaphoreType.DMA,
            pltpu.SemaphoreType.DMA,
        ],
    )(_restore_kernel)
    return f(outputs, indices)
